# Initial kernel scaffold; baseline (speedup 1.0000x reference)
import jax
import jax.numpy as jnp
from jax import lax
from jax.experimental import pallas as pl
from jax.experimental.pallas import tpu as pltpu
from jax.experimental.pallas import tpu_sc as plsc
import functools

RANK = 32
NC = 2
NS = 16
NW = NC * NS
LANES = 16
GRP = 128
CHUNK = 512


def _factor_kernel(n_total):
    n_per_w = n_total // NW
    n_chunks = n_per_w // CHUNK
    jrows = CHUNK // GRP

    mesh = plsc.VectorSubcoreMesh(core_axis_name="c", subcore_axis_name="s")

    @functools.partial(
        pl.kernel,
        out_type=jax.ShapeDtypeStruct((n_total,), jnp.float32),
        mesh=mesh,
        scratch_types=[
            pltpu.VMEM((jrows, GRP), jnp.int32),
            pltpu.VMEM((jrows, GRP), jnp.int32),
            pltpu.VMEM((CHUNK, RANK), jnp.float32),
            pltpu.VMEM((CHUNK, RANK), jnp.float32),
            pltpu.VMEM((CHUNK,), jnp.float32),
            pltpu.SemaphoreType.DMA,
        ],
    )
    def kern(rid_hbm, cid_hbm, u_hbm, v_hbm, out_hbm, ridx, cidx, urows,
             vrows, obuf, sem):
        wid = lax.axis_index("s") * NC + lax.axis_index("c")
        base_row = wid * (n_per_w // GRP)

        def chunk_body(g, _):
            row0 = base_row + g * jrows
            pltpu.sync_copy(rid_hbm.at[pl.ds(row0, jrows)], ridx)
            pltpu.sync_copy(cid_hbm.at[pl.ds(row0, jrows)], cidx)
            for j in range(jrows):
                pltpu.async_copy(
                    u_hbm.at[ridx.at[j]],
                    urows.at[pl.ds(j * GRP, GRP)], sem)
                pltpu.async_copy(
                    v_hbm.at[cidx.at[j]],
                    vrows.at[pl.ds(j * GRP, GRP)], sem)
            for j in range(jrows):
                pltpu.make_async_copy(
                    u_hbm.at[ridx.at[j]],
                    urows.at[pl.ds(j * GRP, GRP)], sem).wait()
                pltpu.make_async_copy(
                    v_hbm.at[cidx.at[j]],
                    vrows.at[pl.ds(j * GRP, GRP)], sem).wait()

            def dot_body(i, _):
                for c in range(LANES):
                    r = i * LANES + c
                    u0 = urows[r, pl.ds(0, LANES)]
                    u1 = urows[r, pl.ds(LANES, LANES)]
                    v0 = vrows[r, pl.ds(0, LANES)]
                    v1 = vrows[r, pl.ds(LANES, LANES)]
                    obuf[r] = jnp.sum(u0 * v0 + u1 * v1)
                return ()

            lax.fori_loop(0, CHUNK // LANES, dot_body, (), unroll=False)
            pltpu.sync_copy(
                obuf, out_hbm.at[pl.ds(wid * n_per_w + g * CHUNK, CHUNK)])
            return ()

        lax.fori_loop(0, n_chunks, chunk_body, (), unroll=False)

    return kern


def kernel(row_ids, col_ids, U, V):
    b, l = row_ids.shape
    n = b * l
    rid = row_ids.reshape(n // GRP, GRP)
    cid = col_ids.reshape(n // GRP, GRP)
    out = _factor_kernel(n)(rid, cid, U, V)
    return out.reshape(b, l)


# speedup vs baseline: 2.5008x; 2.5008x over previous
import jax
import jax.numpy as jnp
from jax import lax
from jax.experimental import pallas as pl
from jax.experimental.pallas import tpu as pltpu
from jax.experimental.pallas import tpu_sc as plsc
import functools

RANK = 32
NC = 2
NS = 16
NW = NC * NS
LANES = 16
GRP = 128
CHUNK = 512

_BITREV = [int(f"{l:04b}"[::-1], 2) for l in range(16)]

_GATHER_DNUMS = lax.GatherDimensionNumbers(
    offset_dims=(), collapsed_slice_dims=(0,), start_index_map=(0,))


def _lane_shuffle(x, idx):
    return lax.gather(
        x, idx[:, None], _GATHER_DNUMS, slice_sizes=(1,),
        mode=lax.GatherScatterMode.PROMISE_IN_BOUNDS)


def _factor_kernel(n_total):
    n_per_w = n_total // NW
    n_chunks = n_per_w // CHUNK
    jrows = CHUNK // GRP

    mesh = plsc.VectorSubcoreMesh(core_axis_name="c", subcore_axis_name="s")

    @functools.partial(
        pl.kernel,
        out_type=jax.ShapeDtypeStruct((n_total,), jnp.float32),
        mesh=mesh,
        scratch_types=[
            pltpu.VMEM((jrows, GRP), jnp.int32),
            pltpu.VMEM((jrows, GRP), jnp.int32),
            pltpu.VMEM((CHUNK, RANK), jnp.float32),
            pltpu.VMEM((CHUNK, RANK), jnp.float32),
            pltpu.VMEM((CHUNK,), jnp.float32),
            pltpu.SemaphoreType.DMA,
        ],
        compiler_params=pltpu.CompilerParams(use_tc_tiling_on_sc=False),
    )
    def kern(rid_hbm, cid_hbm, u_hbm, v_hbm, out_hbm, ridx, cidx, urows,
             vrows, obuf, sem):
        wid = lax.axis_index("s") * NC + lax.axis_index("c")
        base_row = wid * (n_per_w // GRP)

        def chunk_body(g, _):
            row0 = base_row + g * jrows
            pltpu.sync_copy(rid_hbm.at[pl.ds(row0, jrows)], ridx)
            pltpu.sync_copy(cid_hbm.at[pl.ds(row0, jrows)], cidx)
            for j in range(jrows):
                pltpu.async_copy(
                    u_hbm.at[ridx.at[j]],
                    urows.at[pl.ds(j * GRP, GRP)], sem)
                pltpu.async_copy(
                    v_hbm.at[cidx.at[j]],
                    vrows.at[pl.ds(j * GRP, GRP)], sem)
            for j in range(jrows):
                pltpu.make_async_copy(
                    u_hbm.at[ridx.at[j]],
                    urows.at[pl.ds(j * GRP, GRP)], sem).wait()
                pltpu.make_async_copy(
                    v_hbm.at[cidx.at[j]],
                    vrows.at[pl.ds(j * GRP, GRP)], sem).wait()

            iota = lax.broadcasted_iota(jnp.int32, (LANES,), 0)

            def dot_body(i, _):
                ss = []
                for c in range(LANES):
                    r = i * LANES + _BITREV[c]
                    u0 = urows[r, pl.ds(0, LANES)]
                    u1 = urows[r, pl.ds(LANES, LANES)]
                    v0 = vrows[r, pl.ds(0, LANES)]
                    v1 = vrows[r, pl.ds(LANES, LANES)]
                    ss.append(u0 * v0 + u1 * v1)
                for rbit in (8, 4, 2, 1):
                    rot = iota ^ rbit
                    keep = (iota & rbit) == 0
                    nxt = []
                    for k in range(0, len(ss), 2):
                        x, y = ss[k], ss[k + 1]
                        xr = _lane_shuffle(x, rot)
                        yr = _lane_shuffle(y, rot)
                        nxt.append(jnp.where(keep, x + xr, y + yr))
                    ss = nxt
                obuf[pl.ds(i * LANES, LANES)] = ss[0]
                return ()

            lax.fori_loop(0, CHUNK // LANES, dot_body, (), unroll=False)
            pltpu.sync_copy(
                obuf, out_hbm.at[pl.ds(wid * n_per_w + g * CHUNK, CHUNK)])
            return ()

        lax.fori_loop(0, n_chunks, chunk_body, (), unroll=False)

    return kern


def kernel(row_ids, col_ids, U, V):
    b, l = row_ids.shape
    n = b * l
    rid = row_ids.reshape(n // GRP, GRP)
    cid = col_ids.reshape(n // GRP, GRP)
    out = _factor_kernel(n)(rid, cid, U, V)
    return out.reshape(b, l)


# speedup vs baseline: 2.8471x; 1.1385x over previous
import jax
import jax.numpy as jnp
from jax import lax
from jax.experimental import pallas as pl
from jax.experimental.pallas import tpu as pltpu
from jax.experimental.pallas import tpu_sc as plsc
import functools

RANK = 32
NC = 2
NS = 16
NW = NC * NS
LANES = 16
GRP = 128
CHUNK = 512

_BITREV = [int(f"{l:04b}"[::-1], 2) for l in range(16)]

_GATHER_DNUMS = lax.GatherDimensionNumbers(
    offset_dims=(), collapsed_slice_dims=(0,), start_index_map=(0,))


def _lane_shuffle(x, idx):
    return lax.gather(
        x, idx[:, None], _GATHER_DNUMS, slice_sizes=(1,),
        mode=lax.GatherScatterMode.PROMISE_IN_BOUNDS)


def _factor_kernel(n_total):
    n_per_w = n_total // NW
    n_chunks = n_per_w // CHUNK
    jrows = CHUNK // GRP
    idx_rows = n_per_w // GRP

    mesh = plsc.VectorSubcoreMesh(core_axis_name="c", subcore_axis_name="s")

    @functools.partial(
        pl.kernel,
        out_type=jax.ShapeDtypeStruct((n_total,), jnp.float32),
        mesh=mesh,
        scratch_types=[
            pltpu.VMEM((idx_rows, GRP), jnp.int32),
            pltpu.VMEM((idx_rows, GRP), jnp.int32),
            pltpu.VMEM((2, CHUNK, RANK), jnp.float32),
            pltpu.VMEM((2, CHUNK, RANK), jnp.float32),
            pltpu.VMEM((2, CHUNK), jnp.float32),
            pltpu.SemaphoreType.DMA,
            pltpu.SemaphoreType.DMA,
            pltpu.SemaphoreType.DMA,
            pltpu.SemaphoreType.DMA,
        ],
        compiler_params=pltpu.CompilerParams(use_tc_tiling_on_sc=False),
    )
    def kern(rid_hbm, cid_hbm, u_hbm, v_hbm, out_hbm, ridx, cidx, urows,
             vrows, obuf, gsem0, gsem1, osem0, osem1):
        wid = lax.axis_index("s") * NC + lax.axis_index("c")
        base_row = wid * idx_rows
        out_base = wid * n_per_w
        gsems = (gsem0, gsem1)
        osems = (osem0, osem1)
        iota = lax.broadcasted_iota(jnp.int32, (LANES,), 0)

        pltpu.sync_copy(rid_hbm.at[pl.ds(base_row, idx_rows)], ridx)
        pltpu.sync_copy(cid_hbm.at[pl.ds(base_row, idx_rows)], cidx)

        def gather_chunk(g, slot):
            for j in range(jrows):
                pltpu.async_copy(
                    u_hbm.at[ridx.at[g * jrows + j]],
                    urows.at[slot, pl.ds(j * GRP, GRP)], gsems[slot])
                pltpu.async_copy(
                    v_hbm.at[cidx.at[g * jrows + j]],
                    vrows.at[slot, pl.ds(j * GRP, GRP)], gsems[slot])

        def wait_chunk(g, slot):
            for j in range(jrows):
                pltpu.make_async_copy(
                    u_hbm.at[ridx.at[g * jrows + j]],
                    urows.at[slot, pl.ds(j * GRP, GRP)], gsems[slot]).wait()
                pltpu.make_async_copy(
                    v_hbm.at[cidx.at[g * jrows + j]],
                    vrows.at[slot, pl.ds(j * GRP, GRP)], gsems[slot]).wait()

        def out_copy(g, slot):
            return pltpu.make_async_copy(
                obuf.at[slot],
                out_hbm.at[pl.ds(out_base + g * CHUNK, CHUNK)], osems[slot])

        def compute_chunk(g, slot):
            def dot_body(i, _):
                ss = []
                for c in range(LANES):
                    r = i * LANES + _BITREV[c]
                    u0 = urows[slot, r, pl.ds(0, LANES)]
                    u1 = urows[slot, r, pl.ds(LANES, LANES)]
                    v0 = vrows[slot, r, pl.ds(0, LANES)]
                    v1 = vrows[slot, r, pl.ds(LANES, LANES)]
                    ss.append(u0 * v0 + u1 * v1)
                for rbit in (8, 4, 2, 1):
                    rot = iota ^ rbit
                    keep = (iota & rbit) == 0
                    nxt = []
                    for k in range(0, len(ss), 2):
                        x, y = ss[k], ss[k + 1]
                        xr = _lane_shuffle(x, rot)
                        yr = _lane_shuffle(y, rot)
                        nxt.append(jnp.where(keep, x + xr, y + yr))
                    ss = nxt
                obuf[slot, pl.ds(i * LANES, LANES)] = ss[0]
                return ()

            lax.fori_loop(0, CHUNK // LANES, dot_body, (), unroll=False)

        gather_chunk(0, 0)

        def pair_body(i, _):
            for b in range(2):
                g = 2 * i + b
                slot = b

                @pl.when(g + 1 < n_chunks)
                def _():
                    gather_chunk(g + 1, 1 - slot)

                wait_chunk(g, slot)

                @pl.when(g >= 2)
                def _():
                    out_copy(g - 2, slot).wait()

                compute_chunk(g, slot)
                out_copy(g, slot).start()
            return ()

        lax.fori_loop(0, n_chunks // 2, pair_body, (), unroll=False)
        out_copy(n_chunks - 2, 0).wait()
        out_copy(n_chunks - 1, 1).wait()

    return kern


def kernel(row_ids, col_ids, U, V):
    b, l = row_ids.shape
    n = b * l
    rid = row_ids.reshape(n // GRP, GRP)
    cid = col_ids.reshape(n // GRP, GRP)
    out = _factor_kernel(n)(rid, cid, U, V)
    return out.reshape(b, l)
